# Initial kernel scaffold; baseline (speedup 1.0000x reference)
import jax
import jax.numpy as jnp
from jax.experimental import pallas as pl


def kernel(dividend, divisor):
    raise NotImplementedError("write your pallas kernel here")


# speedup vs baseline: 1.8060x; 1.8060x over previous
import jax
import jax.numpy as jnp
from jax.experimental import pallas as pl


def _gaines_div_kernel(d_ref, o_ref):
    o_ref[...] = (d_ref[0] + d_ref[1] > 0.0).astype(jnp.float32)


def kernel(dividend, divisor):
    del divisor
    _, rows, cols = dividend.shape
    block_rows = 256
    grid = (rows // block_rows,)
    return pl.pallas_call(
        _gaines_div_kernel,
        grid=grid,
        in_specs=[pl.BlockSpec((2, block_rows, cols), lambda i: (0, i, 0))],
        out_specs=pl.BlockSpec((block_rows, cols), lambda i: (i, 0)),
        out_shape=jax.ShapeDtypeStruct((rows, cols), jnp.float32),
    )(dividend)


# speedup vs baseline: 1.8078x; 1.0010x over previous
import jax
import jax.numpy as jnp
from jax.experimental import pallas as pl


def _gaines_div_kernel(d_ref, o_ref):
    o_ref[...] = (d_ref[0] + d_ref[1] > 0.0).astype(jnp.float32)


def kernel(dividend, divisor):
    del divisor
    _, rows, cols = dividend.shape
    block_rows = 512
    grid = (rows // block_rows,)
    return pl.pallas_call(
        _gaines_div_kernel,
        grid=grid,
        in_specs=[pl.BlockSpec((2, block_rows, cols), lambda i: (0, i, 0))],
        out_specs=pl.BlockSpec((block_rows, cols), lambda i: (i, 0)),
        out_shape=jax.ShapeDtypeStruct((rows, cols), jnp.float32),
    )(dividend)
